# Initial kernel scaffold; baseline (speedup 1.0000x reference)
import jax
import jax.numpy as jnp
from jax.experimental import pallas as pl


def kernel(x, router_scale, per_expert_scale, W_router, gate_up_proj, down_proj):
    raise NotImplementedError("write your pallas kernel here")


# speedup vs baseline: 1.4130x; 1.4130x over previous
import functools

import jax
import jax.numpy as jnp
from jax import lax
from jax.experimental import pallas as pl
from jax.experimental.pallas import tpu as pltpu
from jax.experimental.pallas import tpu_sc as plsc

HIDDEN = 2048
EXPERT_DIM = 1024
NUM_EXPERTS = 8
TOP_K = 2
EPS_RMS = 1e-6
N_TOKENS = 2048

BLK = 256
NUM_ASSIGN = N_TOKENS * TOP_K
NUM_BLOCKS = NUM_ASSIGN // BLK + NUM_EXPERTS - 1
NPAD = NUM_BLOCKS * BLK

SC_CORES = 2
SC_SUBCORES = 16
SC_WORKERS = SC_CORES * SC_SUBCORES
SC_LANES = 16
W_COLS = 128


def _gelu_tanh(x):
    c = jnp.sqrt(2.0 / jnp.pi).astype(jnp.float32)
    return 0.5 * x * (1.0 + jnp.tanh(c * (x + 0.044715 * x * x * x)))


def _router_body(x_ref, rs_ref, pes_ref, wr_ref, idx_ref, w_ref):
    xb = x_ref[...]
    var = jnp.mean(xb * xb, axis=1, keepdims=True)
    ri = xb * lax.rsqrt(var + EPS_RMS)
    ri = ri * rs_ref[...] * (HIDDEN ** -0.5)
    logits = lax.dot_general(
        ri, wr_ref[...], (((1,), (1,)), ((), ())),
        preferred_element_type=jnp.float32,
        precision=lax.Precision.HIGHEST,
    )
    m = jnp.max(logits, axis=1, keepdims=True)
    p = jnp.exp(logits - m)
    p = p / jnp.sum(p, axis=1, keepdims=True)

    iota8 = lax.broadcasted_iota(jnp.int32, logits.shape, 1)
    i1 = jnp.argmax(logits, axis=1).astype(jnp.int32)
    mask1 = iota8 == i1[:, None]
    neg = jnp.where(mask1, -jnp.inf, logits)
    i2 = jnp.argmax(neg, axis=1).astype(jnp.int32)
    mask2 = iota8 == i2[:, None]

    pes = pes_ref[...]
    w1 = jnp.sum(jnp.where(mask1, p, 0.0), axis=1)
    w2 = jnp.sum(jnp.where(mask2, p, 0.0), axis=1)
    s = w1 + w2
    w1 = w1 / s * jnp.sum(jnp.where(mask1, pes, 0.0), axis=1)
    w2 = w2 / s * jnp.sum(jnp.where(mask2, pes, 0.0), axis=1)

    idx_ref[...] = jnp.where(iota8 == 0, i1[:, None],
                             jnp.where(iota8 == 1, i2[:, None], 0))
    w_ref[...] = jnp.where(iota8 == 0, w1[:, None],
                           jnp.where(iota8 == 1, w2[:, None], 0.0))


def _router(x, router_scale, per_expert_scale, W_router):
    blk_t = 256
    grid = (N_TOKENS // blk_t,)
    return pl.pallas_call(
        _router_body,
        grid=grid,
        in_specs=[
            pl.BlockSpec((blk_t, HIDDEN), lambda i: (i, 0)),
            pl.BlockSpec((1, HIDDEN), lambda i: (0, 0)),
            pl.BlockSpec((1, NUM_EXPERTS), lambda i: (0, 0)),
            pl.BlockSpec((NUM_EXPERTS, HIDDEN), lambda i: (0, 0)),
        ],
        out_specs=[
            pl.BlockSpec((blk_t, NUM_EXPERTS), lambda i: (i, 0)),
            pl.BlockSpec((blk_t, NUM_EXPERTS), lambda i: (i, 0)),
        ],
        out_shape=[
            jax.ShapeDtypeStruct((N_TOKENS, NUM_EXPERTS), jnp.int32),
            jax.ShapeDtypeStruct((N_TOKENS, NUM_EXPERTS), jnp.float32),
        ],
    )(x, router_scale.reshape(1, HIDDEN),
      per_expert_scale.reshape(1, NUM_EXPERTS), W_router)


def _routing_metadata(idxs, ws):
    i1 = idxs[:, 0]
    i2 = idxs[:, 1]
    e_ids = jnp.stack([i1, i2], axis=1).reshape(-1)
    oh = (e_ids[:, None] == jnp.arange(NUM_EXPERTS)[None, :]).astype(jnp.int32)
    csum = jnp.cumsum(oh, axis=0)
    counts = csum[-1]
    rank = jnp.take_along_axis(csum, e_ids[:, None], axis=1)[:, 0] - 1
    padded = ((counts + BLK - 1) // BLK) * BLK
    cum_end = jnp.cumsum(padded)
    aoff = cum_end - padded
    positions = (aoff[e_ids] + rank).astype(jnp.int32)
    pos0 = positions[0::2]
    pos1 = positions[1::2]
    bstart = jnp.arange(NUM_BLOCKS, dtype=jnp.int32) * BLK
    block_expert = jnp.minimum(
        jnp.searchsorted(cum_end, bstart, side="right").astype(jnp.int32),
        NUM_EXPERTS - 1)
    w0_rows = jnp.broadcast_to(ws[:, 0:1], (N_TOKENS, W_COLS))
    w1_rows = jnp.broadcast_to(ws[:, 1:2], (N_TOKENS, W_COLS))
    return pos0, pos1, block_expert, w0_rows, w1_rows


def _sc_scatter(x, pos0, pos1, w0_rows, w1_rows):
    mesh = plsc.VectorSubcoreMesh(core_axis_name="c", subcore_axis_name="s")
    tw = N_TOKENS // SC_WORKERS
    chunk = 16
    nchunks = tw // chunk

    @functools.partial(
        pl.kernel, mesh=mesh,
        out_type=(
            jax.ShapeDtypeStruct((NPAD, HIDDEN), jnp.float32),
            jax.ShapeDtypeStruct((NPAD, W_COLS), jnp.float32),
        ),
        scratch_types=[
            pltpu.VMEM((chunk, HIDDEN), jnp.float32),
            pltpu.VMEM((chunk, W_COLS), jnp.float32),
            pltpu.VMEM((chunk, W_COLS), jnp.float32),
            pltpu.VMEM((chunk,), jnp.int32),
            pltpu.VMEM((chunk,), jnp.int32),
            pltpu.SemaphoreType.DMA,
            pltpu.SemaphoreType.DMA,
            pltpu.SemaphoreType.DMA,
            pltpu.SemaphoreType.DMA,
        ],
    )
    def k(x_hbm, p0_hbm, p1_hbm, w0_hbm, w1_hbm, xs_hbm, ws_hbm,
          xbuf, wbuf0, wbuf1, idx0, idx1, s0, s1, s2, s3):
        wid = lax.axis_index("s") * SC_CORES + lax.axis_index("c")
        base = wid * tw

        @pl.loop(0, nchunks)
        def _(ci):
            tb = base + ci * chunk
            pltpu.sync_copy(p0_hbm.at[pl.ds(tb, chunk)], idx0)
            pltpu.sync_copy(p1_hbm.at[pl.ds(tb, chunk)], idx1)
            pltpu.sync_copy(x_hbm.at[pl.ds(tb, chunk)], xbuf)
            pltpu.sync_copy(w0_hbm.at[pl.ds(tb, chunk)], wbuf0)
            pltpu.sync_copy(w1_hbm.at[pl.ds(tb, chunk)], wbuf1)
            c0 = pltpu.async_copy(xbuf, xs_hbm.at[idx0], s0)
            c1 = pltpu.async_copy(xbuf, xs_hbm.at[idx1], s1)
            c2 = pltpu.async_copy(wbuf0, ws_hbm.at[idx0], s2)
            c3 = pltpu.async_copy(wbuf1, ws_hbm.at[idx1], s3)
            c0.wait()
            c1.wait()
            c2.wait()
            c3.wait()

    return k(x, pos0, pos1, w0_rows, w1_rows)


def _k1_body(be_ref, x_ref, wg_ref, wu_ref, ws_ref, h_ref):
    del be_ref
    xb = x_ref[...].astype(jnp.bfloat16)
    wg = wg_ref[0].astype(jnp.bfloat16)
    wu = wu_ref[0].astype(jnp.bfloat16)
    gate = lax.dot_general(xb, wg, (((1,), (1,)), ((), ())),
                           preferred_element_type=jnp.float32)
    up = lax.dot_general(xb, wu, (((1,), (1,)), ((), ())),
                         preferred_element_type=jnp.float32)
    h_ref[...] = _gelu_tanh(gate) * up * ws_ref[...][:, 0:1]


def _k1(block_expert, xs, gate_up_proj, ws):
    grid_spec = pltpu.PrefetchScalarGridSpec(
        num_scalar_prefetch=1,
        grid=(NUM_BLOCKS,),
        in_specs=[
            pl.BlockSpec((BLK, HIDDEN), lambda g, be: (g, 0)),
            pl.BlockSpec((1, EXPERT_DIM, HIDDEN), lambda g, be: (be[g], 0, 0)),
            pl.BlockSpec((1, EXPERT_DIM, HIDDEN), lambda g, be: (be[g], 1, 0)),
            pl.BlockSpec((BLK, W_COLS), lambda g, be: (g, 0)),
        ],
        out_specs=pl.BlockSpec((BLK, EXPERT_DIM), lambda g, be: (g, 0)),
    )
    return pl.pallas_call(
        _k1_body,
        grid_spec=grid_spec,
        out_shape=jax.ShapeDtypeStruct((NPAD, EXPERT_DIM), jnp.float32),
    )(block_expert, xs, gate_up_proj, gate_up_proj, ws)


def _k2_body(be_ref, h_ref, wd_ref, y_ref):
    del be_ref
    hb = h_ref[...].astype(jnp.bfloat16)
    wd = wd_ref[0].astype(jnp.bfloat16)
    y_ref[...] = lax.dot_general(hb, wd, (((1,), (1,)), ((), ())),
                                 preferred_element_type=jnp.float32)


def _k2(block_expert, h, down_proj):
    grid_spec = pltpu.PrefetchScalarGridSpec(
        num_scalar_prefetch=1,
        grid=(NUM_BLOCKS,),
        in_specs=[
            pl.BlockSpec((BLK, EXPERT_DIM), lambda g, be: (g, 0)),
            pl.BlockSpec((1, HIDDEN, EXPERT_DIM), lambda g, be: (be[g], 0, 0)),
        ],
        out_specs=pl.BlockSpec((BLK, HIDDEN), lambda g, be: (g, 0)),
    )
    return pl.pallas_call(
        _k2_body,
        grid_spec=grid_spec,
        out_shape=jax.ShapeDtypeStruct((NPAD, HIDDEN), jnp.float32),
    )(block_expert, h, down_proj)


def _sc_combine(ys, pos0, pos1):
    mesh = plsc.VectorSubcoreMesh(core_axis_name="c", subcore_axis_name="s")
    tw = N_TOKENS // SC_WORKERS
    chunk = 8
    nchunks = tw // chunk
    cols = HIDDEN // SC_LANES

    @functools.partial(
        pl.kernel, mesh=mesh,
        out_type=jax.ShapeDtypeStruct((N_TOKENS, HIDDEN), jnp.float32),
        scratch_types=[
            pltpu.VMEM((chunk, HIDDEN), jnp.float32),
            pltpu.VMEM((chunk, HIDDEN), jnp.float32),
            pltpu.VMEM((chunk,), jnp.int32),
            pltpu.VMEM((chunk,), jnp.int32),
            pltpu.SemaphoreType.DMA,
            pltpu.SemaphoreType.DMA,
        ],
    )
    def k(ys_hbm, p0_hbm, p1_hbm, out_hbm, buf0, buf1, idx0, idx1, s0, s1):
        wid = lax.axis_index("s") * SC_CORES + lax.axis_index("c")
        base = wid * tw

        @pl.loop(0, nchunks)
        def _(ci):
            tb = base + ci * chunk
            pltpu.sync_copy(p0_hbm.at[pl.ds(tb, chunk)], idx0)
            pltpu.sync_copy(p1_hbm.at[pl.ds(tb, chunk)], idx1)
            c0 = pltpu.async_copy(ys_hbm.at[idx0], buf0, s0)
            c1 = pltpu.async_copy(ys_hbm.at[idx1], buf1, s1)
            c0.wait()
            c1.wait()

            @pl.loop(0, chunk)
            def _(r):
                @pl.loop(0, cols)
                def _(cc):
                    sl = pl.ds(cc * SC_LANES, SC_LANES)
                    buf0[r, sl] = buf0[r, sl] + buf1[r, sl]

            pltpu.sync_copy(buf0, out_hbm.at[pl.ds(tb, chunk)])

    return k(ys, pos0, pos1)


def kernel(x, router_scale, per_expert_scale, W_router, gate_up_proj,
           down_proj):
    _ = _router
    var = jnp.mean(jnp.square(x), axis=-1, keepdims=True)
    router_input = x * lax.rsqrt(var + 1e-6)
    router_input = router_input * router_scale * (HIDDEN ** -0.5)
    router_logits = (router_input @ W_router.T).astype(jnp.float32)
    router_probs = jax.nn.softmax(router_logits, axis=-1)
    _, topk_index = lax.top_k(router_logits, TOP_K)
    topk_weights = jnp.take_along_axis(router_probs, topk_index, axis=-1)
    topk_weights = topk_weights / jnp.sum(topk_weights, axis=-1, keepdims=True)
    topk_weights = topk_weights * per_expert_scale[topk_index]
    idxs = topk_index.astype(jnp.int32)
    ws = topk_weights.astype(jnp.float32)
    pos0, pos1, block_expert, w0_rows, w1_rows = _routing_metadata(idxs, ws)
    xs, wsort = _sc_scatter(x, pos0, pos1, w0_rows, w1_rows)
    h = _k1(block_expert, xs, gate_up_proj, wsort)
    ys = _k2(block_expert, h, down_proj)
    return _sc_combine(ys, pos0, pos1)
